# baseline (speedup 1.0000x reference)
import functools

import jax
import jax.numpy as jnp
from jax import lax
from jax.experimental import pallas as pl
from jax.experimental.pallas import tpu as pltpu
from jax.experimental.pallas import tpu_sc as plsc

N_NODES = 10000
NP = 10240
N_EDGES = 320000
IN_DIM = 128
HID_DIM = 64
NUM_REL = 2

NC = 2
NS = 16
NW = NC * NS

EDGE_B = 128
E_PAD = 327680
K_PER_W = E_PAD // (NW * EDGE_B)
NBUF = 2
ACC_ROWS = NUM_REL * NP
ROWS_PER_TILE = ACC_ROWS // NS
_EB = E_PAD // 128

NB = 512
_NG = NP // NB


def _mm_body(x_ref, wrel_ref, wroot_ref, b_ref, t_ref, r_ref):
    x = x_ref[...]
    t_ref[0] = jnp.dot(x, wrel_ref[0], preferred_element_type=jnp.float32)
    t_ref[1] = jnp.dot(x, wrel_ref[1], preferred_element_type=jnp.float32)
    r_ref[...] = (jnp.dot(x, wroot_ref[...], preferred_element_type=jnp.float32)
                  + b_ref[...][None, :])


_mm_call = pl.pallas_call(
    _mm_body,
    grid=(_NG,),
    in_specs=[
        pl.BlockSpec((NB, IN_DIM), lambda i: (i, 0)),
        pl.BlockSpec((NUM_REL, IN_DIM, HID_DIM), lambda i: (0, 0, 0)),
        pl.BlockSpec((IN_DIM, HID_DIM), lambda i: (0, 0)),
        pl.BlockSpec((HID_DIM,), lambda i: (0,)),
    ],
    out_specs=(
        pl.BlockSpec((NUM_REL, NB, HID_DIM), lambda i: (0, i, 0)),
        pl.BlockSpec((NB, HID_DIM), lambda i: (i, 0)),
    ),
    out_shape=(
        jax.ShapeDtypeStruct((NUM_REL, NP, HID_DIM), jnp.float32),
        jax.ShapeDtypeStruct((NP, HID_DIM), jnp.float32),
    ),
)


def _edge_body(ei_ref, et_ref, gidx_ref, sidx_ref):
    et = et_ref[...]
    gidx_ref[...] = et * NP + ei_ref[0]
    sidx_ref[...] = et * NP + ei_ref[1]


_edge_call = pl.pallas_call(
    _edge_body,
    out_shape=(
        jax.ShapeDtypeStruct((_EB, 128), jnp.int32),
        jax.ShapeDtypeStruct((_EB, 128), jnp.int32),
    ),
)


def _layer_norm(h, g, b):
    mu = jnp.mean(h, axis=-1, keepdims=True)
    var = jnp.mean((h - mu) ** 2, axis=-1, keepdims=True)
    return (h - mu) / jnp.sqrt(var + 1e-5) * g + b


def _combine(acc_ref, cnt_ref):
    c0 = jnp.maximum(cnt_ref[0, 0, :, 0:1] + cnt_ref[1, 0, :, 0:1], 1.0)
    c1 = jnp.maximum(cnt_ref[0, 1, :, 0:1] + cnt_ref[1, 1, :, 0:1], 1.0)
    return ((acc_ref[0, 0] + acc_ref[1, 0]) / c0
            + (acc_ref[0, 1] + acc_ref[1, 1]) / c1)


def _mid_body(r1_ref, acc_ref, cnt_ref, g_ref, bln_ref, wrel_ref, wroot_ref,
              b_ref, t_ref, r_ref):
    h = r1_ref[...] + _combine(acc_ref, cnt_ref)
    h = _layer_norm(h, g_ref[...][None, :], bln_ref[...][None, :])
    h = 0.5 * h * (1.0 + lax.erf(h * (2.0 ** -0.5)))
    t_ref[0] = jnp.dot(h, wrel_ref[0], preferred_element_type=jnp.float32)
    t_ref[1] = jnp.dot(h, wrel_ref[1], preferred_element_type=jnp.float32)
    r_ref[...] = (jnp.dot(h, wroot_ref[...], preferred_element_type=jnp.float32)
                  + b_ref[...][None, :])


_acc_spec = pl.BlockSpec((NC, NUM_REL, NB, HID_DIM), lambda i: (0, 0, i, 0))
_cnt_spec = pl.BlockSpec((NC, NUM_REL, NB, 8), lambda i: (0, 0, i, 0))
_vec_spec = pl.BlockSpec((HID_DIM,), lambda i: (0,))
_row_spec = pl.BlockSpec((NB, HID_DIM), lambda i: (i, 0))

_mid_call = pl.pallas_call(
    _mid_body,
    grid=(_NG,),
    in_specs=[
        _row_spec,
        _acc_spec,
        _cnt_spec,
        _vec_spec,
        _vec_spec,
        pl.BlockSpec((NUM_REL, HID_DIM, HID_DIM), lambda i: (0, 0, 0)),
        pl.BlockSpec((HID_DIM, HID_DIM), lambda i: (0, 0)),
        _vec_spec,
    ],
    out_specs=(
        pl.BlockSpec((NUM_REL, NB, HID_DIM), lambda i: (0, i, 0)),
        _row_spec,
    ),
    out_shape=(
        jax.ShapeDtypeStruct((NUM_REL, NP, HID_DIM), jnp.float32),
        jax.ShapeDtypeStruct((NP, HID_DIM), jnp.float32),
    ),
)


def _final_body(r2_ref, acc_ref, cnt_ref, g_ref, bln_ref, out_ref):
    h = r2_ref[...] + _combine(acc_ref, cnt_ref)
    out_ref[...] = _layer_norm(h, g_ref[...][None, :], bln_ref[...][None, :])


_final_call = pl.pallas_call(
    _final_body,
    grid=(_NG,),
    in_specs=[_row_spec, _acc_spec, _cnt_spec, _vec_spec, _vec_spec],
    out_specs=_row_spec,
    out_shape=jax.ShapeDtypeStruct((NP, HID_DIM), jnp.float32),
)


_SC_MESH = plsc.VectorSubcoreMesh(core_axis_name="c", subcore_axis_name="s")


def _sc_scatter_body(with_cnt, *refs):
    if with_cnt:
        (t_hbm, gidx_hbm, sidx_hbm, z64_hbm, z8_hbm, ones_hbm,
         acc_hbm, cnt_hbm, gidx_v, sidx_v, rows_v, ones_v, acc_sh, cnt_sh,
         *gsems) = refs
    else:
        (t_hbm, gidx_hbm, sidx_hbm, z64_hbm,
         acc_hbm, gidx_v, sidx_v, rows_v, acc_sh, *gsems) = refs
    c = lax.axis_index("c")
    s = lax.axis_index("s")
    w = c * NS + s

    pltpu.sync_copy(gidx_hbm.at[w], gidx_v)
    pltpu.sync_copy(sidx_hbm.at[w], sidx_v)
    for b in range(NBUF):
        pltpu.async_copy(t_hbm.at[gidx_v.at[b]], rows_v.at[b], gsems[b])
    pltpu.sync_copy(z64_hbm, acc_sh.at[pl.ds(s * ROWS_PER_TILE, ROWS_PER_TILE)])
    if with_cnt:
        pltpu.sync_copy(z8_hbm, cnt_sh.at[pl.ds(s * ROWS_PER_TILE, ROWS_PER_TILE)])
        pltpu.sync_copy(ones_hbm, ones_v)
    plsc.subcore_barrier()

    @pl.loop(0, K_PER_W, step=NBUF)
    def _grp(g):
        for b in range(NBUF):
            j = g + b
            pltpu.make_async_copy(t_hbm.at[gidx_v.at[j]], rows_v.at[b],
                                  gsems[b]).wait()
            pltpu.sync_copy(rows_v.at[b], acc_sh.at[sidx_v.at[j]], add=True)
            if with_cnt:
                pltpu.sync_copy(ones_v, cnt_sh.at[sidx_v.at[j]], add=True)

            @pl.when(j + NBUF < K_PER_W)
            def _refill():
                pltpu.async_copy(t_hbm.at[gidx_v.at[j + NBUF]], rows_v.at[b],
                                 gsems[b])

    plsc.subcore_barrier()
    sl = pl.ds(s * ROWS_PER_TILE, ROWS_PER_TILE)
    pltpu.sync_copy(acc_sh.at[sl], acc_hbm.at[c, sl])
    if with_cnt:
        pltpu.sync_copy(cnt_sh.at[sl], cnt_hbm.at[c, sl])


_sc_scatter_cnt = pl.kernel(
    functools.partial(_sc_scatter_body, True),
    out_type=(
        jax.ShapeDtypeStruct((NC, ACC_ROWS, HID_DIM), jnp.float32),
        jax.ShapeDtypeStruct((NC, ACC_ROWS, 8), jnp.float32),
    ),
    mesh=_SC_MESH,
    scratch_types=[
        pltpu.VMEM((K_PER_W, EDGE_B), jnp.int32),
        pltpu.VMEM((K_PER_W, EDGE_B), jnp.int32),
        pltpu.VMEM((NBUF, EDGE_B, HID_DIM), jnp.float32),
        pltpu.VMEM((EDGE_B, 8), jnp.float32),
        pltpu.VMEM_SHARED((ACC_ROWS, HID_DIM), jnp.float32),
        pltpu.VMEM_SHARED((ACC_ROWS, 8), jnp.float32),
    ] + [pltpu.SemaphoreType.DMA] * NBUF,
    compiler_params=pltpu.CompilerParams(use_tc_tiling_on_sc=False),
)

_sc_scatter_nocnt = pl.kernel(
    functools.partial(_sc_scatter_body, False),
    out_type=jax.ShapeDtypeStruct((NC, ACC_ROWS, HID_DIM), jnp.float32),
    mesh=_SC_MESH,
    scratch_types=[
        pltpu.VMEM((K_PER_W, EDGE_B), jnp.int32),
        pltpu.VMEM((K_PER_W, EDGE_B), jnp.int32),
        pltpu.VMEM((NBUF, EDGE_B, HID_DIM), jnp.float32),
        pltpu.VMEM_SHARED((ACC_ROWS, HID_DIM), jnp.float32),
    ] + [pltpu.SemaphoreType.DMA] * NBUF,
    compiler_params=pltpu.CompilerParams(use_tc_tiling_on_sc=False),
)


def kernel(x, edge_index, edge_type, W_rel1, W_root1, b1, ln1_g, ln1_b,
           W_rel2, W_root2, b2, ln2_g, ln2_b):
    xp = jnp.pad(x, ((0, NP - N_NODES), (0, 0)))
    ei = jnp.pad(edge_index, ((0, 0), (0, E_PAD - N_EDGES)),
                 constant_values=N_NODES).reshape(2, _EB, 128)
    et = jnp.pad(edge_type, ((0, E_PAD - N_EDGES),)).reshape(_EB, 128)

    t1, r1 = _mm_call(xp, W_rel1, W_root1, b1)
    gidx, sidx = _edge_call(ei, et)
    t1 = t1.reshape(ACC_ROWS, HID_DIM)
    gidx = gidx.reshape(NW, K_PER_W, EDGE_B)
    sidx = sidx.reshape(NW, K_PER_W, EDGE_B)

    z64 = jnp.zeros((ROWS_PER_TILE, HID_DIM), jnp.float32)
    z8 = jnp.zeros((ROWS_PER_TILE, 8), jnp.float32)
    ones8 = jnp.ones((EDGE_B, 8), jnp.float32)

    acc1, cnt = _sc_scatter_cnt(t1, gidx, sidx, z64, z8, ones8)
    acc1 = acc1.reshape(NC, NUM_REL, NP, HID_DIM)
    cnt = cnt.reshape(NC, NUM_REL, NP, 8)

    t2, r2 = _mid_call(r1, acc1, cnt, ln1_g, ln1_b, W_rel2, W_root2, b2)
    t2 = t2.reshape(ACC_ROWS, HID_DIM)

    acc2 = _sc_scatter_nocnt(t2, gidx, sidx, z64)
    acc2 = acc2.reshape(NC, NUM_REL, NP, HID_DIM)

    out = _final_call(r2, acc2, cnt, ln2_g, ln2_b)
    return out[:N_NODES]


# speedup vs baseline: 2.0276x; 2.0276x over previous
import functools

import jax
import jax.numpy as jnp
from jax import lax
from jax.experimental import pallas as pl
from jax.experimental.pallas import tpu as pltpu
from jax.experimental.pallas import tpu_sc as plsc

N_NODES = 10000
NP = 10240
N_EDGES = 320000
IN_DIM = 128
HID_DIM = 64
NUM_REL = 2

NC = 2
NS = 16
NW = NC * NS

EDGE_B = 128
E_PAD = 327680
K_PER_W = E_PAD // (NW * EDGE_B)
NBUF = 2
ACC_ROWS = NUM_REL * NP
ROWS_PER_TILE = ACC_ROWS // NS
_EB = E_PAD // 128

NB = 512
_NG = NP // NB


def _mm_body(x_ref, wrel_ref, wroot_ref, b_ref, t_ref, r_ref):
    x = x_ref[...]
    t_ref[0] = jnp.dot(x, wrel_ref[0], preferred_element_type=jnp.float32)
    t_ref[1] = jnp.dot(x, wrel_ref[1], preferred_element_type=jnp.float32)
    r_ref[...] = (jnp.dot(x, wroot_ref[...], preferred_element_type=jnp.float32)
                  + b_ref[...][None, :])


_mm_call = pl.pallas_call(
    _mm_body,
    grid=(_NG,),
    in_specs=[
        pl.BlockSpec((NB, IN_DIM), lambda i: (i, 0)),
        pl.BlockSpec((NUM_REL, IN_DIM, HID_DIM), lambda i: (0, 0, 0)),
        pl.BlockSpec((IN_DIM, HID_DIM), lambda i: (0, 0)),
        pl.BlockSpec((HID_DIM,), lambda i: (0,)),
    ],
    out_specs=(
        pl.BlockSpec((NUM_REL, NB, HID_DIM), lambda i: (0, i, 0)),
        pl.BlockSpec((NB, HID_DIM), lambda i: (i, 0)),
    ),
    out_shape=(
        jax.ShapeDtypeStruct((NUM_REL, NP, HID_DIM), jnp.float32),
        jax.ShapeDtypeStruct((NP, HID_DIM), jnp.float32),
    ),
)


def _edge_body(ei_ref, et_ref, gidx_ref, sidx_ref):
    et = et_ref[...]
    gidx_ref[...] = et * NP + ei_ref[0]
    sidx_ref[...] = et * NP + ei_ref[1]


_edge_call = pl.pallas_call(
    _edge_body,
    out_shape=(
        jax.ShapeDtypeStruct((_EB, 128), jnp.int32),
        jax.ShapeDtypeStruct((_EB, 128), jnp.int32),
    ),
)


def _layer_norm(h, g, b):
    mu = jnp.mean(h, axis=-1, keepdims=True)
    var = jnp.mean((h - mu) ** 2, axis=-1, keepdims=True)
    return (h - mu) / jnp.sqrt(var + 1e-5) * g + b


def _combine(acc_ref, cnt_ref):
    c0 = jnp.maximum(cnt_ref[0, 0, :, 0:1] + cnt_ref[1, 0, :, 0:1], 1.0)
    c1 = jnp.maximum(cnt_ref[0, 1, :, 0:1] + cnt_ref[1, 1, :, 0:1], 1.0)
    return ((acc_ref[0, 0] + acc_ref[1, 0]) / c0
            + (acc_ref[0, 1] + acc_ref[1, 1]) / c1)


def _mid_body(r1_ref, acc_ref, cnt_ref, g_ref, bln_ref, wrel_ref, wroot_ref,
              b_ref, t_ref, r_ref):
    h = r1_ref[...] + _combine(acc_ref, cnt_ref)
    h = _layer_norm(h, g_ref[...][None, :], bln_ref[...][None, :])
    h = 0.5 * h * (1.0 + lax.erf(h * (2.0 ** -0.5)))
    t_ref[0] = jnp.dot(h, wrel_ref[0], preferred_element_type=jnp.float32)
    t_ref[1] = jnp.dot(h, wrel_ref[1], preferred_element_type=jnp.float32)
    r_ref[...] = (jnp.dot(h, wroot_ref[...], preferred_element_type=jnp.float32)
                  + b_ref[...][None, :])


_acc_spec = pl.BlockSpec((NC, NUM_REL, NB, HID_DIM), lambda i: (0, 0, i, 0))
_cnt_spec = pl.BlockSpec((NC, NUM_REL, NB, 8), lambda i: (0, 0, i, 0))
_vec_spec = pl.BlockSpec((HID_DIM,), lambda i: (0,))
_row_spec = pl.BlockSpec((NB, HID_DIM), lambda i: (i, 0))

_mid_call = pl.pallas_call(
    _mid_body,
    grid=(_NG,),
    in_specs=[
        _row_spec,
        _acc_spec,
        _cnt_spec,
        _vec_spec,
        _vec_spec,
        pl.BlockSpec((NUM_REL, HID_DIM, HID_DIM), lambda i: (0, 0, 0)),
        pl.BlockSpec((HID_DIM, HID_DIM), lambda i: (0, 0)),
        _vec_spec,
    ],
    out_specs=(
        pl.BlockSpec((NUM_REL, NB, HID_DIM), lambda i: (0, i, 0)),
        _row_spec,
    ),
    out_shape=(
        jax.ShapeDtypeStruct((NUM_REL, NP, HID_DIM), jnp.float32),
        jax.ShapeDtypeStruct((NP, HID_DIM), jnp.float32),
    ),
)


def _final_body(r2_ref, acc_ref, cnt_ref, g_ref, bln_ref, out_ref):
    h = r2_ref[...] + _combine(acc_ref, cnt_ref)
    out_ref[...] = _layer_norm(h, g_ref[...][None, :], bln_ref[...][None, :])


_final_call = pl.pallas_call(
    _final_body,
    grid=(_NG,),
    in_specs=[_row_spec, _acc_spec, _cnt_spec, _vec_spec, _vec_spec],
    out_specs=_row_spec,
    out_shape=jax.ShapeDtypeStruct((NP, HID_DIM), jnp.float32),
)


_SC_MESH = plsc.VectorSubcoreMesh(core_axis_name="c", subcore_axis_name="s")


def _sc_scatter_body(with_cnt, *refs):
    if with_cnt:
        (t_hbm, gidx_hbm, sidx_hbm, z64_hbm, z8_hbm, ones_hbm,
         acc_hbm, cnt_hbm, gidx_v, sidx_v, rows_v, ones_v, acc_sh, cnt_sh,
         *gsems) = refs
    else:
        (t_hbm, gidx_hbm, sidx_hbm, z64_hbm,
         acc_hbm, gidx_v, sidx_v, rows_v, acc_sh, *gsems) = refs
    c = lax.axis_index("c")
    s = lax.axis_index("s")
    w = c * NS + s

    pltpu.sync_copy(gidx_hbm.at[w], gidx_v)
    pltpu.sync_copy(sidx_hbm.at[w], sidx_v)
    for b in range(NBUF):
        pltpu.async_copy(t_hbm.at[gidx_v.at[b]], rows_v.at[b], gsems[b])
    pltpu.sync_copy(z64_hbm, acc_sh.at[pl.ds(s * ROWS_PER_TILE, ROWS_PER_TILE)])
    if with_cnt:
        pltpu.sync_copy(z8_hbm, cnt_sh.at[pl.ds(s * ROWS_PER_TILE, ROWS_PER_TILE)])
        pltpu.sync_copy(ones_hbm, ones_v)
    plsc.subcore_barrier()

    @pl.loop(0, K_PER_W, step=NBUF)
    def _grp(g):
        for b in range(NBUF):
            j = g + b
            pltpu.make_async_copy(t_hbm.at[gidx_v.at[j]], rows_v.at[b],
                                  gsems[b]).wait()
            pltpu.sync_copy(rows_v.at[b], acc_sh.at[sidx_v.at[j]], add=True)
            if with_cnt:
                pltpu.sync_copy(ones_v, cnt_sh.at[sidx_v.at[j]], add=True)

            @pl.when(j + NBUF < K_PER_W)
            def _refill():
                pltpu.async_copy(t_hbm.at[gidx_v.at[j + NBUF]], rows_v.at[b],
                                 gsems[b])

    plsc.subcore_barrier()
    sl = pl.ds(s * ROWS_PER_TILE, ROWS_PER_TILE)
    pltpu.sync_copy(acc_sh.at[sl], acc_hbm.at[c, sl])
    if with_cnt:
        pltpu.sync_copy(cnt_sh.at[sl], cnt_hbm.at[c, sl])


_sc_scatter_cnt = pl.kernel(
    functools.partial(_sc_scatter_body, True),
    out_type=(
        jax.ShapeDtypeStruct((NC, ACC_ROWS, HID_DIM), jnp.float32),
        jax.ShapeDtypeStruct((NC, ACC_ROWS, 8), jnp.float32),
    ),
    mesh=_SC_MESH,
    scratch_types=[
        pltpu.VMEM((K_PER_W, EDGE_B), jnp.int32),
        pltpu.VMEM((K_PER_W, EDGE_B), jnp.int32),
        pltpu.VMEM((NBUF, EDGE_B, HID_DIM), jnp.float32),
        pltpu.VMEM((EDGE_B, 8), jnp.float32),
        pltpu.VMEM_SHARED((ACC_ROWS, HID_DIM), jnp.float32),
        pltpu.VMEM_SHARED((ACC_ROWS, 8), jnp.float32),
    ] + [pltpu.SemaphoreType.DMA] * NBUF,
    compiler_params=pltpu.CompilerParams(use_tc_tiling_on_sc=False),
)

_sc_scatter_nocnt = pl.kernel(
    functools.partial(_sc_scatter_body, False),
    out_type=jax.ShapeDtypeStruct((NC, ACC_ROWS, HID_DIM), jnp.float32),
    mesh=_SC_MESH,
    scratch_types=[
        pltpu.VMEM((K_PER_W, EDGE_B), jnp.int32),
        pltpu.VMEM((K_PER_W, EDGE_B), jnp.int32),
        pltpu.VMEM((NBUF, EDGE_B, HID_DIM), jnp.float32),
        pltpu.VMEM_SHARED((ACC_ROWS, HID_DIM), jnp.float32),
    ] + [pltpu.SemaphoreType.DMA] * NBUF,
    compiler_params=pltpu.CompilerParams(use_tc_tiling_on_sc=False),
)


def kernel(x, edge_index, edge_type, W_rel1, W_root1, b1, ln1_g, ln1_b,
           W_rel2, W_root2, b2, ln2_g, ln2_b):
    xp = jnp.pad(x, ((0, NP - N_NODES), (0, 0)))
    pad_idx = N_NODES + (jnp.arange(E_PAD - N_EDGES, dtype=jnp.int32)
                         % (NP - N_NODES))
    ei = jnp.concatenate(
        [edge_index, jnp.stack([pad_idx, pad_idx])], axis=1).reshape(2, _EB, 128)
    et = jnp.concatenate(
        [edge_type, jnp.zeros_like(pad_idx)]).reshape(_EB, 128)

    t1, r1 = _mm_call(xp, W_rel1, W_root1, b1)
    gidx, sidx = _edge_call(ei, et)
    t1 = t1.reshape(ACC_ROWS, HID_DIM)
    gidx = gidx.reshape(NW, K_PER_W, EDGE_B)
    sidx = sidx.reshape(NW, K_PER_W, EDGE_B)

    z64 = jnp.zeros((ROWS_PER_TILE, HID_DIM), jnp.float32)
    z8 = jnp.zeros((ROWS_PER_TILE, 8), jnp.float32)
    ones8 = jnp.ones((EDGE_B, 8), jnp.float32)

    acc1, cnt = _sc_scatter_cnt(t1, gidx, sidx, z64, z8, ones8)
    acc1 = acc1.reshape(NC, NUM_REL, NP, HID_DIM)
    cnt = cnt.reshape(NC, NUM_REL, NP, 8)

    t2, r2 = _mid_call(r1, acc1, cnt, ln1_g, ln1_b, W_rel2, W_root2, b2)
    t2 = t2.reshape(ACC_ROWS, HID_DIM)

    acc2 = _sc_scatter_nocnt(t2, gidx, sidx, z64)
    acc2 = acc2.reshape(NC, NUM_REL, NP, HID_DIM)

    out = _final_call(r2, acc2, cnt, ln2_g, ln2_b)
    return out[:N_NODES]
